# Initial kernel scaffold; baseline (speedup 1.0000x reference)
import functools

import jax
import jax.numpy as jnp
from jax import lax
from jax.experimental import pallas as pl
from jax.experimental.pallas import tpu as pltpu
from jax.experimental.pallas import tpu_sc as plsc

N = 10000
D = 128
E = 320000

NC = 2
NS = 16
NW = NC * NS
EPW = E // NW
B = 80
NB = EPW // B
ACC_ROWS = 10240
ZROWS = ACC_ROWS // NS
OROWS = N // NS


def _sc_body(x_hbm, src_hbm, dst_hbm, w_hbm, out_hbm,
             src_v, dst_v, w_v, rows_v, acc_sh, sem):
    c = lax.axis_index("c")
    s = lax.axis_index("s")
    wid = s * NC + c

    zero = jnp.zeros((16,), jnp.float32)

    def zfill(i, carry):
        for k in range(8):
            rows_v[i, pl.ds(k * 16, 16)] = zero
        return carry

    lax.fori_loop(0, B, zfill, 0)

    for q in range(ZROWS // B):
        pltpu.sync_copy(rows_v.at[pl.ds(0, B)],
                        acc_sh.at[pl.ds(s * ZROWS + q * B, B)])
    plsc.subcore_barrier()

    ebase = wid * EPW

    def batch(j, carry):
        e0 = ebase + j * B
        pltpu.sync_copy(src_hbm.at[pl.ds(e0, B)], src_v)
        pltpu.sync_copy(dst_hbm.at[pl.ds(e0, B)], dst_v)
        pltpu.sync_copy(w_hbm.at[pl.ds(e0, B)], w_v)
        pltpu.async_copy(x_hbm.at[src_v], rows_v, sem).wait()

        def scale(i, carry2):
            w = w_v[i]
            for k in range(8):
                sl = pl.ds(k * 16, 16)
                rows_v[i, sl] = rows_v[i, sl] * w
            return carry2

        lax.fori_loop(0, B, scale, 0)
        pltpu.sync_copy(rows_v, acc_sh.at[dst_v], add=True)
        return carry

    lax.fori_loop(0, NB, batch, 0)

    plsc.subcore_barrier()
    pltpu.sync_copy(acc_sh.at[pl.ds(s * OROWS, OROWS)],
                    out_hbm.at[c, pl.ds(s * OROWS, OROWS)])


@jax.jit
def _sc_spmm(x, src, dst, w):
    mesh = plsc.VectorSubcoreMesh(core_axis_name="c", subcore_axis_name="s")
    return pl.kernel(
        _sc_body,
        out_type=jax.ShapeDtypeStruct((NC, N, D), jnp.float32),
        mesh=mesh,
        scratch_types=[
            pltpu.VMEM((B,), jnp.int32),
            pltpu.VMEM((B,), jnp.int32),
            pltpu.VMEM((B,), jnp.float32),
            pltpu.VMEM((B, D), jnp.float32),
            pltpu.VMEM_SHARED((ACC_ROWS, D), jnp.float32),
            pltpu.SemaphoreType.DMA,
        ],
    )(x, src, dst, w)


def _tc_body(p0_ref, p1_ref, w_ref, b_ref, o_ref):
    agg = p0_ref[...] + p1_ref[...]
    o_ref[...] = lax.dot_general(
        agg, w_ref[...], (((1,), (1,)), ((), ())),
        preferred_element_type=jnp.float32) + b_ref[...]


@jax.jit
def _tc_combine(p0, p1, W, b2d):
    bm = 1250
    grid = (N // bm,)
    return pl.pallas_call(
        _tc_body,
        grid=grid,
        in_specs=[
            pl.BlockSpec((bm, D), lambda i: (i, 0)),
            pl.BlockSpec((bm, D), lambda i: (i, 0)),
            pl.BlockSpec((D, D), lambda i: (0, 0)),
            pl.BlockSpec((1, D), lambda i: (0, 0)),
        ],
        out_specs=pl.BlockSpec((bm, D), lambda i: (i, 0)),
        out_shape=jax.ShapeDtypeStruct((N, D), jnp.float32),
    )(p0, p1, W, b2d)


def kernel(input_feature, edge_index, edge_weight, W, b):
    src = edge_index[0]
    dst = edge_index[1]
    partials = _sc_spmm(input_feature, src, dst, edge_weight)
    return _tc_combine(partials[0], partials[1], W, b.reshape(1, D))


# speedup vs baseline: 4.1448x; 4.1448x over previous
import functools

import jax
import jax.numpy as jnp
from jax import lax
from jax.experimental import pallas as pl
from jax.experimental.pallas import tpu as pltpu
from jax.experimental.pallas import tpu_sc as plsc

N = 10000
D = 128
E = 320000

NC = 2
NS = 16
NW = NC * NS
EPW = E // NW
B = 80
NB = EPW // B
ACC_ROWS = 10240
ZROWS = ACC_ROWS // NS


def _sc_body(x_hbm, src_hbm, dst_hbm, w_hbm, out_hbm,
             src_v, dst_v, w_v, rows_v, acc_sh, sem):
    c = lax.axis_index("c")
    s = lax.axis_index("s")
    wid = s * NC + c

    zero = jnp.zeros((16,), jnp.float32)

    def zfill(i, carry):
        for k in range(8):
            rows_v[i, pl.ds(k * 16, 16)] = zero
        return carry

    lax.fori_loop(0, B, zfill, 0)

    for q in range(ZROWS // B):
        pltpu.sync_copy(rows_v.at[pl.ds(0, B)],
                        acc_sh.at[pl.ds(s * ZROWS + q * B, B)])
    plsc.subcore_barrier()

    ebase = wid * EPW

    def batch(j, carry):
        e0 = ebase + j * B
        pltpu.sync_copy(src_hbm.at[pl.ds(e0, B)], src_v)
        pltpu.sync_copy(dst_hbm.at[pl.ds(e0, B)], dst_v)
        pltpu.sync_copy(w_hbm.at[pl.ds(e0, B)], w_v)
        pltpu.async_copy(x_hbm.at[src_v], rows_v, sem).wait()

        def scale(g, carry2):
            wv = w_v[pl.ds(g * 16, 16)]
            for e in range(16):
                w = wv[e]
                i = g * 16 + e
                for k in range(8):
                    sl = pl.ds(k * 16, 16)
                    rows_v[i, sl] = rows_v[i, sl] * w
            return carry2

        lax.fori_loop(0, B // 16, scale, 0)
        pltpu.sync_copy(rows_v, acc_sh.at[dst_v], add=True)
        return carry

    lax.fori_loop(0, NB, batch, 0)

    plsc.subcore_barrier()
    pltpu.sync_copy(acc_sh.at[pl.ds(s * ZROWS, ZROWS)],
                    out_hbm.at[c, pl.ds(s * ZROWS, ZROWS)])


@jax.jit
def _sc_spmm(x, src, dst, w):
    mesh = plsc.VectorSubcoreMesh(core_axis_name="c", subcore_axis_name="s")
    return pl.kernel(
        _sc_body,
        out_type=jax.ShapeDtypeStruct((NC, ACC_ROWS, D), jnp.float32),
        mesh=mesh,
        scratch_types=[
            pltpu.VMEM((B,), jnp.int32),
            pltpu.VMEM((B,), jnp.int32),
            pltpu.VMEM((B,), jnp.float32),
            pltpu.VMEM((B, D), jnp.float32),
            pltpu.VMEM_SHARED((ACC_ROWS, D), jnp.float32),
            pltpu.SemaphoreType.DMA,
        ],
    )(x, src, dst, w)


def _tc_body(p0_ref, p1_ref, w_ref, b_ref, o_ref):
    agg = p0_ref[...] + p1_ref[...]
    o_ref[...] = lax.dot_general(
        agg, w_ref[...], (((1,), (1,)), ((), ())),
        preferred_element_type=jnp.float32) + b_ref[...]


@jax.jit
def _tc_combine(p0, p1, W, b2d):
    bm = 2000
    grid = (N // bm,)
    return pl.pallas_call(
        _tc_body,
        grid=grid,
        in_specs=[
            pl.BlockSpec((bm, D), lambda i: (i, 0)),
            pl.BlockSpec((bm, D), lambda i: (i, 0)),
            pl.BlockSpec((D, D), lambda i: (0, 0)),
            pl.BlockSpec((1, D), lambda i: (0, 0)),
        ],
        out_specs=pl.BlockSpec((bm, D), lambda i: (i, 0)),
        out_shape=jax.ShapeDtypeStruct((N, D), jnp.float32),
    )(p0, p1, W, b2d)


def kernel(input_feature, edge_index, edge_weight, W, b):
    src = edge_index[0]
    dst = edge_index[1]
    partials = _sc_spmm(input_feature, src, dst, edge_weight)
    return _tc_combine(partials[0, :N], partials[1, :N], W, b.reshape(1, D))
